# baseline (speedup 1.0000x reference)
import functools

import jax
import jax.numpy as jnp
from jax.experimental import pallas as pl
from jax.experimental.pallas import tpu as pltpu
from jax.experimental.pallas import tpu_sc as plsc

_T2 = 1024
_K = 3
_L = 16


def _knn_kernel(xyz1_ref, flow1_ref, xyz2_ref, out_ref):
    keys = xyz1_ref[0] + flow1_ref[0]
    q = xyz2_ref[0]

    mm = jax.lax.dot_general(
        (-2.0 * keys).astype(jnp.bfloat16), q.astype(jnp.bfloat16),
        dimension_numbers=(((0,), (0,)), ((), ())),
        preferred_element_type=jnp.float32,
    )
    q2 = q[0] * q[0] + q[1] * q[1] + q[2] * q[2]
    k2 = keys[0] * keys[0] + keys[1] * keys[1] + keys[2] * keys[2]
    d_sel = (mm + q2[None, :]) + k2[:, None]

    iota = jax.lax.broadcasted_iota(jnp.int32, d_sel.shape, 0)
    bits = jax.lax.bitcast_convert_type(d_sel, jnp.int32)
    u = jax.lax.bitcast_convert_type(
        (bits & jnp.int32(~0xFFF)) | iota, jnp.float32)

    inf = jnp.float32(jnp.inf)
    m1 = jnp.min(u, axis=0)
    m2 = jnp.min(jnp.where(u <= m1[None, :], inf, u), axis=0)
    m3 = jnp.min(jnp.where(u <= m2[None, :], inf, u), axis=0)

    out_ref[0, 0, :] = m1
    out_ref[0, 1, :] = m2
    out_ref[0, 2, :] = m3


def _rsqrt_newton(dd):
    dd = jnp.maximum(dd, jnp.float32(1e-24))
    y = jax.lax.bitcast_convert_type(
        jnp.int32(0x5F3759DF)
        - jax.lax.shift_right_arithmetic(
            jax.lax.bitcast_convert_type(dd, jnp.int32), 1),
        jnp.float32)
    half = jnp.float32(0.5) * dd
    for _ in range(3):
        y = y * (jnp.float32(1.5) - half * y * y)
    return y


def _warp_sc(nc, n1, n2, qpw, boff,
             x1h, y1h, z1h, fxh, fyh, fzh, qxh, qyh, qzh,
             m1h, m2h, m3h,
             oxh, oyh, ozh,
             x1v, y1v, z1v, fxv, fyv, fzv,
             qxv, qyv, qzv, m1v, m2v, m3v, oxv, oyv, ozv):
    wid = jax.lax.axis_index("s") * nc + jax.lax.axis_index("c")
    wpb = n2 // qpw
    bi = wid // wpb
    pbi = bi + boff
    qbase = (wid % wpb) * qpw
    mvs = (m1v, m2v, m3v)
    if True:
        pltpu.sync_copy(x1h.at[pl.ds(pbi * n1, n1)], x1v)
        pltpu.sync_copy(y1h.at[pl.ds(pbi * n1, n1)], y1v)
        pltpu.sync_copy(z1h.at[pl.ds(pbi * n1, n1)], z1v)
        pltpu.sync_copy(fxh.at[pl.ds(pbi * n1, n1)], fxv)
        pltpu.sync_copy(fyh.at[pl.ds(pbi * n1, n1)], fyv)
        pltpu.sync_copy(fzh.at[pl.ds(pbi * n1, n1)], fzv)
        qfull = pl.ds(pbi * n2 + qbase, qpw)
        qsl = pl.ds(bi * n2 + qbase, qpw)
        pltpu.sync_copy(qxh.at[qfull], qxv)
        pltpu.sync_copy(qyh.at[qfull], qyv)
        pltpu.sync_copy(qzh.at[qfull], qzv)
        pltpu.sync_copy(m1h.at[qsl], m1v)
        pltpu.sync_copy(m2h.at[qsl], m2v)
        pltpu.sync_copy(m3h.at[qsl], m3v)

        for g in range(qpw // _L):
            sl = pl.ds(g * _L, _L)
            qx = qxv[sl]
            qy = qyv[sl]
            qz = qzv[sl]
            inv_sum = jnp.zeros((_L,), jnp.float32)
            acc = [jnp.zeros((_L,), jnp.float32) for _ in range(3)]
            for k in range(_K):
                idx = (jax.lax.bitcast_convert_type(mvs[k][sl], jnp.int32)
                       & jnp.int32(0xFFF))
                gx = plsc.load_gather(x1v, [idx])
                gy = plsc.load_gather(y1v, [idx])
                gz = plsc.load_gather(z1v, [idx])
                fx = plsc.load_gather(fxv, [idx])
                fy = plsc.load_gather(fyv, [idx])
                fz = plsc.load_gather(fzv, [idx])
                dx = (gx + fx) - qx
                dy = (gy + fy) - qy
                dz = (gz + fz) - qz
                dd = dx * dx + dy * dy + dz * dz
                inv = jnp.minimum(_rsqrt_newton(dd), jnp.float32(1e10))
                inv_sum = inv_sum + inv
                acc[0] = acc[0] + inv * fx
                acc[1] = acc[1] + inv * fy
                acc[2] = acc[2] + inv * fz
            oxv[sl] = qx - acc[0] / inv_sum
            oyv[sl] = qy - acc[1] / inv_sum
            ozv[sl] = qz - acc[2] / inv_sum

        pltpu.sync_copy(oxv, oxh.at[qsl])
        pltpu.sync_copy(oyv, oyh.at[qsl])
        pltpu.sync_copy(ozv, ozh.at[qsl])


def kernel(xyz1, xyz2, flow1, neighr):
    del neighr
    b, _, n1 = xyz1.shape
    n2 = xyz2.shape[2]

    info = plsc.get_sparse_core_info()
    nw = info.num_cores * info.num_subcores

    x1, y1, z1 = (xyz1[:, c, :].reshape(-1) for c in range(3))
    fx, fy, fz = (flow1[:, c, :].reshape(-1) for c in range(3))
    qx, qy, qz = (xyz2[:, c, :].reshape(-1) for c in range(3))

    halves = 2 if b % 2 == 0 else 1
    bh = b // halves
    qpw = (bh * n2) // nw
    flat = jax.ShapeDtypeStruct((bh * n2,), jnp.float32)
    outs = []
    for h in range(halves):
        boff = h * bh
        m_packed = pl.pallas_call(
            _knn_kernel,
            grid=(bh, n2 // _T2),
            in_specs=[
                pl.BlockSpec((1, 3, n1), lambda bi, j, o=boff: (bi + o, 0, 0)),
                pl.BlockSpec((1, 3, n1), lambda bi, j, o=boff: (bi + o, 0, 0)),
                pl.BlockSpec((1, 3, _T2), lambda bi, j, o=boff: (bi + o, 0, j)),
            ],
            out_specs=pl.BlockSpec((1, 3, _T2), lambda bi, j: (bi, 0, j)),
            out_shape=jax.ShapeDtypeStruct((bh, 3, n2), jnp.float32),
            compiler_params=pltpu.CompilerParams(
                dimension_semantics=("parallel", "parallel"),
            ),
        )(xyz1, flow1, xyz2)
        mm1, mm2, mm3 = (m_packed[:, c, :].reshape(-1) for c in range(3))

        sc = pl.kernel(
            functools.partial(_warp_sc, info.num_cores, n1, n2, qpw, boff),
            out_type=(flat, flat, flat),
            mesh=plsc.VectorSubcoreMesh(
                core_axis_name="c", subcore_axis_name="s"),
            compiler_params=pltpu.CompilerParams(needs_layout_passes=False),
            scratch_types=(
                [pltpu.VMEM((n1,), jnp.float32)] * 6
                + [pltpu.VMEM((qpw,), jnp.float32)] * 9
            ),
        )
        outs.append(sc(x1, y1, z1, fx, fy, fz, qx, qy, qz, mm1, mm2, mm3))

    ox = jnp.concatenate([o[0] for o in outs])
    oy = jnp.concatenate([o[1] for o in outs])
    oz = jnp.concatenate([o[2] for o in outs])
    return jnp.stack(
        [ox.reshape(b, n2), oy.reshape(b, n2), oz.reshape(b, n2)], axis=1)


# speedup vs baseline: 1.0015x; 1.0015x over previous
import functools

import jax
import jax.numpy as jnp
from jax.experimental import pallas as pl
from jax.experimental.pallas import tpu as pltpu
from jax.experimental.pallas import tpu_sc as plsc

_T2 = 1024
_K = 3
_L = 16


def _knn_kernel(xyz1_ref, flow1_ref, xyz2_ref, out_ref):
    keys = xyz1_ref[0] + flow1_ref[0]
    q = xyz2_ref[0]

    mm = jax.lax.dot_general(
        (-2.0 * keys).astype(jnp.bfloat16), q.astype(jnp.bfloat16),
        dimension_numbers=(((0,), (0,)), ((), ())),
        preferred_element_type=jnp.float32,
    )
    q2 = q[0] * q[0] + q[1] * q[1] + q[2] * q[2]
    k2 = keys[0] * keys[0] + keys[1] * keys[1] + keys[2] * keys[2]
    d_sel = (mm + q2[None, :]) + k2[:, None]

    iota = jax.lax.broadcasted_iota(jnp.int32, d_sel.shape, 0)
    bits = jax.lax.bitcast_convert_type(d_sel, jnp.int32)
    u = jax.lax.bitcast_convert_type(
        (bits & jnp.int32(~0xFFF)) | iota, jnp.float32)

    inf = jnp.float32(jnp.inf)
    m1 = jnp.min(u, axis=0)
    m2 = jnp.min(jnp.where(u <= m1[None, :], inf, u), axis=0)
    m3 = jnp.min(jnp.where(u <= m2[None, :], inf, u), axis=0)

    out_ref[0, 0, :] = m1
    out_ref[0, 1, :] = m2
    out_ref[0, 2, :] = m3


def _rsqrt_newton(dd):
    dd = jnp.maximum(dd, jnp.float32(1e-24))
    y = jax.lax.bitcast_convert_type(
        jnp.int32(0x5F3759DF)
        - jax.lax.shift_right_arithmetic(
            jax.lax.bitcast_convert_type(dd, jnp.int32), 1),
        jnp.float32)
    half = jnp.float32(0.5) * dd
    for _ in range(3):
        y = y * (jnp.float32(1.5) - half * y * y)
    return y


def _warp_sc(nc, n1, n2, qpw,
             x1h, y1h, z1h, fxh, fyh, fzh, qxh, qyh, qzh,
             m1h, m2h, m3h,
             oxh, oyh, ozh,
             x1v, y1v, z1v, fxv, fyv, fzv,
             qxv, qyv, qzv, m1v, m2v, m3v, oxv, oyv, ozv):
    wid = jax.lax.axis_index("s") * nc + jax.lax.axis_index("c")
    wpb = n2 // qpw
    bi = wid // wpb
    qbase = (wid % wpb) * qpw
    mvs = (m1v, m2v, m3v)
    if True:
        pltpu.sync_copy(x1h.at[pl.ds(bi * n1, n1)], x1v)
        pltpu.sync_copy(y1h.at[pl.ds(bi * n1, n1)], y1v)
        pltpu.sync_copy(z1h.at[pl.ds(bi * n1, n1)], z1v)
        pltpu.sync_copy(fxh.at[pl.ds(bi * n1, n1)], fxv)
        pltpu.sync_copy(fyh.at[pl.ds(bi * n1, n1)], fyv)
        pltpu.sync_copy(fzh.at[pl.ds(bi * n1, n1)], fzv)
        qsl = pl.ds(bi * n2 + qbase, qpw)
        pltpu.sync_copy(qxh.at[qsl], qxv)
        pltpu.sync_copy(qyh.at[qsl], qyv)
        pltpu.sync_copy(qzh.at[qsl], qzv)
        pltpu.sync_copy(m1h.at[qsl], m1v)
        pltpu.sync_copy(m2h.at[qsl], m2v)
        pltpu.sync_copy(m3h.at[qsl], m3v)

        for g in range(qpw // _L):
            sl = pl.ds(g * _L, _L)
            qx = qxv[sl]
            qy = qyv[sl]
            qz = qzv[sl]
            inv_sum = jnp.zeros((_L,), jnp.float32)
            acc = [jnp.zeros((_L,), jnp.float32) for _ in range(3)]
            for k in range(_K):
                idx = (jax.lax.bitcast_convert_type(mvs[k][sl], jnp.int32)
                       & jnp.int32(0xFFF))
                gx = plsc.load_gather(x1v, [idx])
                gy = plsc.load_gather(y1v, [idx])
                gz = plsc.load_gather(z1v, [idx])
                fx = plsc.load_gather(fxv, [idx])
                fy = plsc.load_gather(fyv, [idx])
                fz = plsc.load_gather(fzv, [idx])
                dx = (gx + fx) - qx
                dy = (gy + fy) - qy
                dz = (gz + fz) - qz
                dd = dx * dx + dy * dy + dz * dz
                inv = jnp.minimum(_rsqrt_newton(dd), jnp.float32(1e10))
                inv_sum = inv_sum + inv
                acc[0] = acc[0] + inv * fx
                acc[1] = acc[1] + inv * fy
                acc[2] = acc[2] + inv * fz
            oxv[sl] = qx - acc[0] / inv_sum
            oyv[sl] = qy - acc[1] / inv_sum
            ozv[sl] = qz - acc[2] / inv_sum

        pltpu.sync_copy(oxv, oxh.at[qsl])
        pltpu.sync_copy(oyv, oyh.at[qsl])
        pltpu.sync_copy(ozv, ozh.at[qsl])


def kernel(xyz1, xyz2, flow1, neighr):
    del neighr
    b, _, n1 = xyz1.shape
    n2 = xyz2.shape[2]

    m_packed = pl.pallas_call(
        _knn_kernel,
        grid=(b, n2 // _T2),
        in_specs=[
            pl.BlockSpec((1, 3, n1), lambda bi, j: (bi, 0, 0)),
            pl.BlockSpec((1, 3, n1), lambda bi, j: (bi, 0, 0)),
            pl.BlockSpec((1, 3, _T2), lambda bi, j: (bi, 0, j)),
        ],
        out_specs=pl.BlockSpec((1, 3, _T2), lambda bi, j: (bi, 0, j)),
        out_shape=jax.ShapeDtypeStruct((b, 3, n2), jnp.float32),
        compiler_params=pltpu.CompilerParams(
            dimension_semantics=("parallel", "parallel"),
        ),
    )(xyz1, flow1, xyz2)

    info = plsc.get_sparse_core_info()
    nw = info.num_cores * info.num_subcores
    qpw = (b * n2) // nw

    x1, y1, z1 = (xyz1[:, c, :].reshape(-1) for c in range(3))
    fx, fy, fz = (flow1[:, c, :].reshape(-1) for c in range(3))
    qx, qy, qz = (xyz2[:, c, :].reshape(-1) for c in range(3))
    mm1, mm2, mm3 = (m_packed[:, c, :].reshape(-1) for c in range(3))

    flat = jax.ShapeDtypeStruct((b * n2,), jnp.float32)
    sc = pl.kernel(
        functools.partial(_warp_sc, info.num_cores, n1, n2, qpw),
        out_type=(flat, flat, flat),
        mesh=plsc.VectorSubcoreMesh(core_axis_name="c", subcore_axis_name="s"),
        compiler_params=pltpu.CompilerParams(needs_layout_passes=False),
        scratch_types=(
            [pltpu.VMEM((n1,), jnp.float32)] * 6
            + [pltpu.VMEM((qpw,), jnp.float32)] * 9
        ),
    )
    ox, oy, oz = sc(x1, y1, z1, fx, fy, fz, qx, qy, qz, mm1, mm2, mm3)

    return jnp.stack(
        [ox.reshape(b, n2), oy.reshape(b, n2), oz.reshape(b, n2)], axis=1)
